# Initial kernel scaffold; baseline (speedup 1.0000x reference)
import functools

import jax
import jax.numpy as jnp
from jax import lax
from jax.experimental import pallas as pl
from jax.experimental.pallas import tpu as pltpu
from jax.experimental.pallas import tpu_sc as plsc

DIM = 128
B = 16384
NUM_CARDS = 7
NCHUNK = DIM // 16

_info = plsc.get_sparse_core_info()
NC, NS = _info.num_cores, _info.num_subcores
NW = NC * NS
ROWS_PER_W = B // NW


def _sc_body(inp_hbm, card_hbm, rank_hbm, suit_hbm, out_hbm,
             inp_v, card_v, rank_v, suit_v, comb_v, out_v):
  wid = lax.axis_index("s") * NC + lax.axis_index("c")

  pltpu.sync_copy(card_hbm, card_v)
  pltpu.sync_copy(rank_hbm, rank_v)
  pltpu.sync_copy(suit_hbm, suit_v)
  n_inp = ROWS_PER_W * NUM_CARDS
  pltpu.sync_copy(inp_hbm.at[pl.ds(wid * n_inp, n_inp)], inp_v)

  @pl.loop(0, 52)
  def _build(c):
    r = lax.shift_right_logical(c, 2)
    s = lax.bitwise_and(c, 3)
    cb = c * DIM
    rb = r * DIM
    sb = s * DIM
    for k in range(NCHUNK):
      comb_v[pl.ds(cb + k * 16, 16)] = (
          card_v[pl.ds(cb + k * 16, 16)]
          + rank_v[pl.ds(rb + k * 16, 16)]
          + suit_v[pl.ds(sb + k * 16, 16)])

  @pl.loop(0, ROWS_PER_W)
  def _row(b):
    bases = [inp_v[b * NUM_CARDS + j] * DIM for j in range(NUM_CARDS)]
    ob = b * DIM
    for k in range(NCHUNK):
      acc = comb_v[pl.ds(bases[0] + k * 16, 16)]
      for j in range(1, NUM_CARDS):
        acc = acc + comb_v[pl.ds(bases[j] + k * 16, 16)]
      out_v[pl.ds(ob + k * 16, 16)] = acc

  pltpu.sync_copy(out_v, out_hbm.at[pl.ds(wid * ROWS_PER_W * DIM,
                                          ROWS_PER_W * DIM)])


@jax.jit
def _card_embed(inp_flat, card_flat, rank_flat, suit_flat):
  mesh = plsc.VectorSubcoreMesh(core_axis_name="c", subcore_axis_name="s")
  kern = pl.kernel(
      _sc_body,
      out_type=jax.ShapeDtypeStruct((B * DIM,), jnp.float32),
      mesh=mesh,
      scratch_types=[
          pltpu.VMEM((ROWS_PER_W * NUM_CARDS,), jnp.int32),
          pltpu.VMEM((52 * DIM,), jnp.float32),
          pltpu.VMEM((13 * DIM,), jnp.float32),
          pltpu.VMEM((4 * DIM,), jnp.float32),
          pltpu.VMEM((52 * DIM,), jnp.float32),
          pltpu.VMEM((ROWS_PER_W * DIM,), jnp.float32),
      ],
  )
  return kern(inp_flat, card_flat, rank_flat, suit_flat)


def kernel(input, rank_table, suit_table, card_table):
  inp_flat = input.astype(jnp.int32).reshape(-1)
  out = _card_embed(inp_flat, card_table.reshape(-1),
                    rank_table.reshape(-1), suit_table.reshape(-1))
  return out.reshape(B, DIM)


# speedup vs baseline: 16.9435x; 16.9435x over previous
import functools

import jax
import jax.numpy as jnp
from jax import lax
from jax.experimental import pallas as pl
from jax.experimental.pallas import tpu as pltpu
from jax.experimental.pallas import tpu_sc as plsc

DIM = 128
B = 16384
NUM_CARDS = 7
NCHUNK = DIM // 16

_info = plsc.get_sparse_core_info()
NC, NS = _info.num_cores, _info.num_subcores
NW = NC * NS
ROWS_PER_W = B // NW


def _sc_body(inp_hbm, card_hbm, rank_hbm, suit_hbm, out_hbm,
             inp_v, card_v, rank_v, suit_v, comb_v, out_v):
  wid = lax.axis_index("s") * NC + lax.axis_index("c")

  pltpu.sync_copy(card_hbm, card_v)
  pltpu.sync_copy(rank_hbm, rank_v)
  pltpu.sync_copy(suit_hbm, suit_v)
  n_inp = ROWS_PER_W * NUM_CARDS
  pltpu.sync_copy(inp_hbm.at[pl.ds(wid * n_inp, n_inp)],
                  inp_v.at[pl.ds(0, n_inp)])

  @pl.loop(0, 52)
  def _build(c):
    r = lax.shift_right_logical(c, 2)
    s = lax.bitwise_and(c, 3)
    cb = c * DIM
    rb = r * DIM
    sb = s * DIM
    for k in range(NCHUNK):
      comb_v[pl.ds(cb + k * 16, 16)] = (
          card_v[pl.ds(cb + k * 16, 16)]
          + rank_v[pl.ds(rb + k * 16, 16)]
          + suit_v[pl.ds(sb + k * 16, 16)])

  @pl.loop(0, ROWS_PER_W)
  def _row(b):
    iv = inp_v[pl.ds(b * NUM_CARDS, 16)] * DIM
    bases = [iv[j] for j in range(NUM_CARDS)]
    ob = b * DIM
    for k in range(NCHUNK):
      acc = comb_v[pl.ds(bases[0] + k * 16, 16)]
      for j in range(1, NUM_CARDS):
        acc = acc + comb_v[pl.ds(bases[j] + k * 16, 16)]
      out_v[pl.ds(ob + k * 16, 16)] = acc

  pltpu.sync_copy(out_v, out_hbm.at[pl.ds(wid * ROWS_PER_W * DIM,
                                          ROWS_PER_W * DIM)])


@jax.jit
def _card_embed(inp_flat, card_flat, rank_flat, suit_flat):
  mesh = plsc.VectorSubcoreMesh(core_axis_name="c", subcore_axis_name="s")
  kern = pl.kernel(
      _sc_body,
      out_type=jax.ShapeDtypeStruct((B * DIM,), jnp.float32),
      mesh=mesh,
      scratch_types=[
          pltpu.VMEM((ROWS_PER_W * NUM_CARDS + 16,), jnp.int32),
          pltpu.VMEM((52 * DIM,), jnp.float32),
          pltpu.VMEM((13 * DIM,), jnp.float32),
          pltpu.VMEM((4 * DIM,), jnp.float32),
          pltpu.VMEM((52 * DIM,), jnp.float32),
          pltpu.VMEM((ROWS_PER_W * DIM,), jnp.float32),
      ],
  )
  return kern(inp_flat, card_flat, rank_flat, suit_flat)


def kernel(input, rank_table, suit_table, card_table):
  inp_flat = input.astype(jnp.int32).reshape(-1)
  out = _card_embed(inp_flat, card_table.reshape(-1),
                    rank_table.reshape(-1), suit_table.reshape(-1))
  return out.reshape(B, DIM)


# speedup vs baseline: 27.3133x; 1.6120x over previous
import functools

import jax
import jax.numpy as jnp
from jax import lax
from jax.experimental import pallas as pl
from jax.experimental.pallas import tpu as pltpu
from jax.experimental.pallas import tpu_sc as plsc

DIM = 128
B = 16384
NUM_CARDS = 7
NCHUNK = DIM // 16

_info = plsc.get_sparse_core_info()
NC, NS = _info.num_cores, _info.num_subcores
NW = NC * NS
ROWS_PER_W = B // NW


def _sc_body(inp_hbm, card_hbm, rank_hbm, suit_hbm, out_hbm,
             inp_v, card_v, rank_v, suit_v, comb_v, out_v):
  wid = lax.axis_index("s") * NC + lax.axis_index("c")

  pltpu.sync_copy(card_hbm, card_v)
  pltpu.sync_copy(rank_hbm, rank_v)
  pltpu.sync_copy(suit_hbm, suit_v)
  n_inp = ROWS_PER_W * NUM_CARDS
  pltpu.sync_copy(inp_hbm.at[pl.ds(wid * n_inp, n_inp)],
                  inp_v.at[pl.ds(0, n_inp)])

  @pl.loop(0, 52)
  def _build(c):
    r = lax.shift_right_logical(c, 2)
    s = lax.bitwise_and(c, 3)
    cb = c * DIM
    rb = r * DIM
    sb = s * DIM
    for k in range(NCHUNK):
      comb_v[pl.ds(cb + k * 16, 16)] = (
          card_v[pl.ds(cb + k * 16, 16)]
          + rank_v[pl.ds(rb + k * 16, 16)]
          + suit_v[pl.ds(sb + k * 16, 16)])

  @plsc.parallel_loop(0, ROWS_PER_W, unroll=2)
  def _row(b):
    iv = inp_v[pl.ds(b * NUM_CARDS, 16)] * DIM
    bases = [iv[j] for j in range(NUM_CARDS)]
    ob = b * DIM
    accs = []
    for k in range(NCHUNK):
      g = [comb_v[pl.ds(bases[j] + k * 16, 16)] for j in range(NUM_CARDS)]
      t01 = g[0] + g[1]
      t23 = g[2] + g[3]
      t45 = g[4] + g[5]
      accs.append((t01 + t23) + (t45 + g[6]))
    for k in range(NCHUNK):
      out_v[pl.ds(ob + k * 16, 16)] = accs[k]

  pltpu.sync_copy(out_v, out_hbm.at[pl.ds(wid * ROWS_PER_W * DIM,
                                          ROWS_PER_W * DIM)])


@jax.jit
def _card_embed(inp_flat, card_flat, rank_flat, suit_flat):
  mesh = plsc.VectorSubcoreMesh(core_axis_name="c", subcore_axis_name="s")
  kern = pl.kernel(
      _sc_body,
      out_type=jax.ShapeDtypeStruct((B * DIM,), jnp.float32),
      mesh=mesh,
      scratch_types=[
          pltpu.VMEM((ROWS_PER_W * NUM_CARDS + 16,), jnp.int32),
          pltpu.VMEM((52 * DIM,), jnp.float32),
          pltpu.VMEM((13 * DIM,), jnp.float32),
          pltpu.VMEM((4 * DIM,), jnp.float32),
          pltpu.VMEM((52 * DIM,), jnp.float32),
          pltpu.VMEM((ROWS_PER_W * DIM,), jnp.float32),
      ],
  )
  return kern(inp_flat, card_flat, rank_flat, suit_flat)


def kernel(input, rank_table, suit_table, card_table):
  inp_flat = input.astype(jnp.int32).reshape(-1)
  out = _card_embed(inp_flat, card_table.reshape(-1),
                    rank_table.reshape(-1), suit_table.reshape(-1))
  return out.reshape(B, DIM)
